# baseline (speedup 1.0000x reference)
import jax
import jax.numpy as jnp
from jax import lax
from jax.experimental import pallas as pl
from jax.experimental.pallas import tpu as pltpu
from jax.experimental.pallas import tpu_sc as plsc

BATCH = 16384
X_DIM = 1000
XC_DIM = 64
LANES = 16

NUM_CORES = 2
NUM_SUBCORES = 16
NUM_WORKERS = NUM_CORES * NUM_SUBCORES
ROWS_PER_WORKER = BATCH // NUM_WORKERS
NUM_CHUNKS = ROWS_PER_WORKER // LANES

NUM_STRANDS = 4
COLS_PER_STRAND = X_DIM // NUM_STRANDS

X_STRIDE = X_DIM + 1


def _argmax16(xbuf, lane):
    def col_body(c, carry):
        ms, idxs = carry
        cvec = jnp.broadcast_to(c, (LANES,))
        new_ms = []
        new_idxs = []
        for k in range(NUM_STRANDS):
            ck = cvec + jnp.int32(k * COLS_PER_STRAND)
            v = plsc.load_gather(xbuf, [lane, ck])
            cond = v > ms[k]
            new_ms.append(jnp.where(cond, v, ms[k]))
            new_idxs.append(jnp.where(cond, ck, idxs[k]))
        return tuple(new_ms), tuple(new_idxs)

    init = (
        tuple(jnp.full((LANES,), -jnp.inf, jnp.float32) for _ in range(NUM_STRANDS)),
        tuple(jnp.zeros((LANES,), jnp.int32) for _ in range(NUM_STRANDS)),
    )
    ms, idxs = lax.fori_loop(0, COLS_PER_STRAND, col_body, init, unroll=4)

    m, idx = ms[0], idxs[0]
    for k in range(1, NUM_STRANDS):
        cond = ms[k] > m
        m = jnp.where(cond, ms[k], m)
        idx = jnp.where(cond, idxs[k], idx)
    return idx


def _tec_body(x_hbm, table_hbm, out_hbm, xbuf0, xbuf1, idxbuf, rowsbuf, sem0, sem1, semg):
    wid = lax.axis_index("s") * NUM_CORES + lax.axis_index("c")
    base = wid * ROWS_PER_WORKER
    lane = lax.iota(jnp.int32, LANES)

    def x_slice(g):
        return x_hbm.at[pl.ds(base + g * LANES, LANES)]

    def xbuf_dst(buf):
        return buf.at[:, pl.ds(0, X_DIM)]

    def finish_chunk(g, xbuf):
        idxbuf[...] = _argmax16(xbuf, lane)
        pltpu.async_copy(table_hbm.at[idxbuf], rowsbuf, semg).wait()
        pltpu.sync_copy(rowsbuf, out_hbm.at[pl.ds(base + g * LANES, LANES)])

    pltpu.async_copy(x_slice(0), xbuf_dst(xbuf0), sem0)

    def pair_body(h, _):
        g0 = 2 * h
        pltpu.async_copy(x_slice(g0 + 1), xbuf_dst(xbuf1), sem1)
        pltpu.make_async_copy(x_slice(g0), xbuf_dst(xbuf0), sem0).wait()
        finish_chunk(g0, xbuf0)

        @pl.when(h + 1 < NUM_CHUNKS // 2)
        def _prefetch():
            pltpu.async_copy(x_slice(g0 + 2), xbuf_dst(xbuf0), sem0)

        pltpu.make_async_copy(x_slice(g0 + 1), xbuf_dst(xbuf1), sem1).wait()
        finish_chunk(g0 + 1, xbuf1)
        return _

    lax.fori_loop(0, NUM_CHUNKS // 2, pair_body, None)


@jax.jit
def kernel(x, twohot_table):
    mesh = plsc.VectorSubcoreMesh(core_axis_name="c", subcore_axis_name="s")
    run = pl.kernel(
        _tec_body,
        out_type=jax.ShapeDtypeStruct((BATCH, XC_DIM), jnp.float32),
        mesh=mesh,
        scratch_types=[
            pltpu.VMEM((LANES, X_STRIDE), jnp.float32),
            pltpu.VMEM((LANES, X_STRIDE), jnp.float32),
            pltpu.VMEM((LANES,), jnp.int32),
            pltpu.VMEM((LANES, XC_DIM), jnp.float32),
            pltpu.SemaphoreType.DMA,
            pltpu.SemaphoreType.DMA,
            pltpu.SemaphoreType.DMA,
        ],
        compiler_params=pltpu.CompilerParams(
            use_tc_tiling_on_sc=False, needs_layout_passes=False
        ),
    )
    return run(x, twohot_table)


# speedup vs baseline: 1.1555x; 1.1555x over previous
import jax
import jax.numpy as jnp
from jax import lax
from jax.experimental import pallas as pl
from jax.experimental.pallas import tpu as pltpu
from jax.experimental.pallas import tpu_sc as plsc

BATCH = 16384
X_DIM = 1000
XC_DIM = 64
LANES = 16

NUM_CORES = 2
NUM_SUBCORES = 16
NUM_WORKERS = NUM_CORES * NUM_SUBCORES
ROWS_PER_WORKER = BATCH // NUM_WORKERS
NUM_CHUNKS = ROWS_PER_WORKER // LANES

NUM_STRANDS = 4
COLS_PER_STRAND = X_DIM // NUM_STRANDS

def _argmax16(xbuf, lane):
    def col_body(c, carry):
        ms, idxs = carry
        cvec = jnp.broadcast_to(c, (LANES,))
        new_ms = []
        new_idxs = []
        for k in range(NUM_STRANDS):
            ck = cvec + jnp.int32(k * COLS_PER_STRAND)
            v = plsc.load_gather(xbuf, [lane, ck])
            cond = v > ms[k]
            new_ms.append(jnp.where(cond, v, ms[k]))
            new_idxs.append(jnp.where(cond, ck, idxs[k]))
        return tuple(new_ms), tuple(new_idxs)

    init = (
        tuple(jnp.full((LANES,), -jnp.inf, jnp.float32) for _ in range(NUM_STRANDS)),
        tuple(jnp.zeros((LANES,), jnp.int32) for _ in range(NUM_STRANDS)),
    )
    ms, idxs = lax.fori_loop(0, COLS_PER_STRAND, col_body, init, unroll=4)

    m, idx = ms[0], idxs[0]
    for k in range(1, NUM_STRANDS):
        cond = ms[k] > m
        m = jnp.where(cond, ms[k], m)
        idx = jnp.where(cond, idxs[k], idx)
    return idx


def _tec_body(x_hbm, table_hbm, out_hbm, xbuf0, xbuf1, idxbuf, rowsbuf, sem0, sem1, semg):
    wid = lax.axis_index("s") * NUM_CORES + lax.axis_index("c")
    base = wid * ROWS_PER_WORKER
    lane = lax.iota(jnp.int32, LANES)

    def x_slice(g):
        return x_hbm.at[pl.ds(base + g * LANES, LANES)]

    def finish_chunk(g, xbuf):
        idxbuf[...] = _argmax16(xbuf, lane)
        pltpu.async_copy(table_hbm.at[idxbuf], rowsbuf, semg).wait()
        pltpu.sync_copy(rowsbuf, out_hbm.at[pl.ds(base + g * LANES, LANES)])

    pltpu.async_copy(x_slice(0), xbuf0, sem0)

    def pair_body(h, _):
        g0 = 2 * h
        pltpu.async_copy(x_slice(g0 + 1), xbuf1, sem1)
        pltpu.make_async_copy(x_slice(g0), xbuf0, sem0).wait()
        finish_chunk(g0, xbuf0)

        @pl.when(h + 1 < NUM_CHUNKS // 2)
        def _prefetch():
            pltpu.async_copy(x_slice(g0 + 2), xbuf0, sem0)

        pltpu.make_async_copy(x_slice(g0 + 1), xbuf1, sem1).wait()
        finish_chunk(g0 + 1, xbuf1)
        return _

    lax.fori_loop(0, NUM_CHUNKS // 2, pair_body, None)


@jax.jit
def kernel(x, twohot_table):
    mesh = plsc.VectorSubcoreMesh(core_axis_name="c", subcore_axis_name="s")
    run = pl.kernel(
        _tec_body,
        out_type=jax.ShapeDtypeStruct((BATCH, XC_DIM), jnp.float32),
        mesh=mesh,
        scratch_types=[
            pltpu.VMEM((LANES, X_DIM), jnp.float32),
            pltpu.VMEM((LANES, X_DIM), jnp.float32),
            pltpu.VMEM((LANES,), jnp.int32),
            pltpu.VMEM((LANES, XC_DIM), jnp.float32),
            pltpu.SemaphoreType.DMA,
            pltpu.SemaphoreType.DMA,
            pltpu.SemaphoreType.DMA,
        ],
        compiler_params=pltpu.CompilerParams(
            use_tc_tiling_on_sc=False, needs_layout_passes=False
        ),
    )
    return run(x, twohot_table)


# speedup vs baseline: 1.2568x; 1.0876x over previous
import jax
import jax.numpy as jnp
from jax import lax
from jax.experimental import pallas as pl
from jax.experimental.pallas import tpu as pltpu
from jax.experimental.pallas import tpu_sc as plsc

BATCH = 16384
X_DIM = 1000
XC_DIM = 64
LANES = 16

NUM_CORES = 2
NUM_SUBCORES = 16
NUM_WORKERS = NUM_CORES * NUM_SUBCORES
ROWS_PER_WORKER = BATCH // NUM_WORKERS
NUM_CHUNKS = ROWS_PER_WORKER // LANES

_STARTS = tuple(range(0, X_DIM - LANES, LANES)) + (X_DIM - LANES,)
NUM_STRANDS = 3
_STRANDS = (_STARTS[0:21], _STARTS[21:42], _STARTS[42:63])

T_STRIDE = LANES + 1

_BIG = 2**30


def _row_scan(xbuf, r):
    ms = []
    idxs = []
    for strand in _STRANDS:
        m = jnp.full((LANES,), -jnp.inf, jnp.float32)
        i = jnp.zeros((LANES,), jnp.int32)
        for start in strand:
            v = xbuf[r, pl.ds(start, LANES)]
            cond = v > m
            m = jnp.maximum(m, v)
            i = jnp.where(cond, jnp.int32(start), i)
        ms.append(m)
        idxs.append(i)
    m, i = ms[0], idxs[0]
    for k in range(1, NUM_STRANDS):
        cond = ms[k] > m
        m = jnp.where(cond, ms[k], m)
        i = jnp.where(cond, idxs[k], i)
    return m, i


def _chunk_argmax(xbuf, mbuf, gbuf, idxbuf, lane):
    def row_body(r, _):
        m, i = _row_scan(xbuf, r)
        mbuf[r, pl.ds(0, LANES)] = m
        gbuf[r, pl.ds(0, LANES)] = i + lane
        return _

    lax.fori_loop(0, LANES, row_body, None)

    tms = []
    tgs = []
    for l in range(LANES):
        col = jnp.full((LANES,), l, jnp.int32)
        tms.append(plsc.load_gather(mbuf, [lane, col]))
        tgs.append(plsc.load_gather(gbuf, [lane, col]))
    m = tms[0]
    for l in range(1, LANES):
        m = jnp.maximum(m, tms[l])
    cand = jnp.full((LANES,), _BIG, jnp.int32)
    for l in range(LANES):
        cand = jnp.minimum(cand, jnp.where(tms[l] == m, tgs[l], jnp.int32(_BIG)))
    idxbuf[...] = cand


def _tec_body(x_hbm, table_hbm, out_hbm, xbuf0, xbuf1, mbuf, gbuf, idxbuf,
              rowsbuf, sem0, sem1, semg):
    wid = lax.axis_index("s") * NUM_CORES + lax.axis_index("c")
    base = wid * ROWS_PER_WORKER
    lane = lax.iota(jnp.int32, LANES)

    def x_slice(g):
        return x_hbm.at[pl.ds(base + g * LANES, LANES)]

    def finish_chunk(g, xbuf):
        _chunk_argmax(xbuf, mbuf, gbuf, idxbuf, lane)
        pltpu.async_copy(table_hbm.at[idxbuf], rowsbuf, semg).wait()
        pltpu.sync_copy(rowsbuf, out_hbm.at[pl.ds(base + g * LANES, LANES)])

    pltpu.async_copy(x_slice(0), xbuf0, sem0)

    def pair_body(h, _):
        g0 = 2 * h
        pltpu.async_copy(x_slice(g0 + 1), xbuf1, sem1)
        pltpu.make_async_copy(x_slice(g0), xbuf0, sem0).wait()
        finish_chunk(g0, xbuf0)

        @pl.when(h + 1 < NUM_CHUNKS // 2)
        def _prefetch():
            pltpu.async_copy(x_slice(g0 + 2), xbuf0, sem0)

        pltpu.make_async_copy(x_slice(g0 + 1), xbuf1, sem1).wait()
        finish_chunk(g0 + 1, xbuf1)
        return _

    lax.fori_loop(0, NUM_CHUNKS // 2, pair_body, None)


@jax.jit
def kernel(x, twohot_table):
    mesh = plsc.VectorSubcoreMesh(core_axis_name="c", subcore_axis_name="s")
    run = pl.kernel(
        _tec_body,
        out_type=jax.ShapeDtypeStruct((BATCH, XC_DIM), jnp.float32),
        mesh=mesh,
        scratch_types=[
            pltpu.VMEM((LANES, X_DIM), jnp.float32),
            pltpu.VMEM((LANES, X_DIM), jnp.float32),
            pltpu.VMEM((LANES, T_STRIDE), jnp.float32),
            pltpu.VMEM((LANES, T_STRIDE), jnp.int32),
            pltpu.VMEM((LANES,), jnp.int32),
            pltpu.VMEM((LANES, XC_DIM), jnp.float32),
            pltpu.SemaphoreType.DMA,
            pltpu.SemaphoreType.DMA,
            pltpu.SemaphoreType.DMA,
        ],
        compiler_params=pltpu.CompilerParams(
            use_tc_tiling_on_sc=False, needs_layout_passes=False
        ),
    )
    return run(x, twohot_table)


# speedup vs baseline: 1.3035x; 1.0372x over previous
import jax
import jax.numpy as jnp
from jax import lax
from jax.experimental import pallas as pl
from jax.experimental.pallas import tpu as pltpu
from jax.experimental.pallas import tpu_sc as plsc

BATCH = 16384
X_DIM = 1000
XC_DIM = 64
LANES = 16

NUM_CORES = 2
NUM_SUBCORES = 16
NUM_WORKERS = NUM_CORES * NUM_SUBCORES
ROWS_PER_WORKER = BATCH // NUM_WORKERS
NUM_CHUNKS = ROWS_PER_WORKER // LANES
CHUNKS_PER_SUPER = 8
SUPER_ROWS = CHUNKS_PER_SUPER * LANES

_STARTS = tuple(range(0, X_DIM - LANES, LANES)) + (X_DIM - LANES,)
NUM_STRANDS = 3
_STRANDS = (_STARTS[0:21], _STARTS[21:42], _STARTS[42:63])

T_STRIDE = LANES + 1

_BIG = 2**30

CHUNK_WORDS = LANES * X_DIM


def _row_scan(xbuf, rbase):
    ms = []
    idxs = []
    for strand in _STRANDS:
        m = jnp.full((LANES,), -jnp.inf, jnp.float32)
        i = jnp.zeros((LANES,), jnp.int32)
        for start in strand:
            v = xbuf[pl.ds(rbase + start, LANES)]
            cond = v > m
            m = jnp.maximum(m, v)
            i = jnp.where(cond, jnp.int32(start), i)
        ms.append(m)
        idxs.append(i)
    m, i = ms[0], idxs[0]
    for k in range(1, NUM_STRANDS):
        cond = ms[k] > m
        m = jnp.where(cond, ms[k], m)
        i = jnp.where(cond, idxs[k], i)
    return m, i


def _chunk_argmax(xbuf, mbuf, gbuf, lane):
    def row_body(r, _):
        m, i = _row_scan(xbuf, r * X_DIM)
        mbuf[r, pl.ds(0, LANES)] = m
        gbuf[r, pl.ds(0, LANES)] = i + lane
        return _

    lax.fori_loop(0, LANES, row_body, None)

    tms = []
    tgs = []
    for l in range(LANES):
        col = jnp.full((LANES,), l, jnp.int32)
        tms.append(plsc.load_gather(mbuf, [lane, col]))
        tgs.append(plsc.load_gather(gbuf, [lane, col]))
    m = tms[0]
    for l in range(1, LANES):
        m = jnp.maximum(m, tms[l])
    cand = jnp.full((LANES,), _BIG, jnp.int32)
    for l in range(LANES):
        cand = jnp.minimum(cand, jnp.where(tms[l] == m, tgs[l], jnp.int32(_BIG)))
    return cand


def _tec_body(x_hbm, table_hbm, out_hbm, xbuf0, xbuf1, mbuf, gbuf, idxbuf,
              rowsbuf, sem0, sem1, semg):
    wid = lax.axis_index("s") * NUM_CORES + lax.axis_index("c")
    base = wid * ROWS_PER_WORKER
    lane = lax.iota(jnp.int32, LANES)

    def x_slice(g):
        return x_hbm.at[pl.ds((base + g * LANES) * X_DIM, CHUNK_WORDS)]

    def finish_chunk(g, xbuf):
        idx16 = _chunk_argmax(xbuf, mbuf, gbuf, lane)
        q = lax.rem(g, CHUNKS_PER_SUPER)
        idxbuf[pl.ds(q * LANES, LANES)] = idx16

        @pl.when(q == CHUNKS_PER_SUPER - 1)
        def _flush():
            pltpu.async_copy(table_hbm.at[idxbuf], rowsbuf, semg).wait()
            row0 = base + (g + 1 - CHUNKS_PER_SUPER) * LANES
            pltpu.sync_copy(rowsbuf, out_hbm.at[pl.ds(row0, SUPER_ROWS)])

    pltpu.async_copy(x_slice(0), xbuf0, sem0)

    def pair_body(h, _):
        g0 = 2 * h
        pltpu.async_copy(x_slice(g0 + 1), xbuf1, sem1)
        pltpu.make_async_copy(x_slice(g0), xbuf0, sem0).wait()
        finish_chunk(g0, xbuf0)

        @pl.when(h + 1 < NUM_CHUNKS // 2)
        def _prefetch():
            pltpu.async_copy(x_slice(g0 + 2), xbuf0, sem0)

        pltpu.make_async_copy(x_slice(g0 + 1), xbuf1, sem1).wait()
        finish_chunk(g0 + 1, xbuf1)
        return _

    lax.fori_loop(0, NUM_CHUNKS // 2, pair_body, None)


@jax.jit
def kernel(x, twohot_table):
    mesh = plsc.VectorSubcoreMesh(core_axis_name="c", subcore_axis_name="s")
    run = pl.kernel(
        _tec_body,
        out_type=jax.ShapeDtypeStruct((BATCH, XC_DIM), jnp.float32),
        mesh=mesh,
        scratch_types=[
            pltpu.VMEM((CHUNK_WORDS,), jnp.float32),
            pltpu.VMEM((CHUNK_WORDS,), jnp.float32),
            pltpu.VMEM((LANES, T_STRIDE), jnp.float32),
            pltpu.VMEM((LANES, T_STRIDE), jnp.int32),
            pltpu.VMEM((SUPER_ROWS,), jnp.int32),
            pltpu.VMEM((SUPER_ROWS, XC_DIM), jnp.float32),
            pltpu.SemaphoreType.DMA,
            pltpu.SemaphoreType.DMA,
            pltpu.SemaphoreType.DMA,
        ],
        compiler_params=pltpu.CompilerParams(
            use_tc_tiling_on_sc=False, needs_layout_passes=False
        ),
    )
    return run(x.reshape(-1), twohot_table)


# speedup vs baseline: 1.3306x; 1.0208x over previous
import jax
import jax.numpy as jnp
from jax import lax
from jax.experimental import pallas as pl
from jax.experimental.pallas import tpu as pltpu
from jax.experimental.pallas import tpu_sc as plsc

BATCH = 16384
X_DIM = 1000
XC_DIM = 64
LANES = 16

NUM_CORES = 2
NUM_SUBCORES = 16
NUM_WORKERS = NUM_CORES * NUM_SUBCORES
ROWS_PER_WORKER = BATCH // NUM_WORKERS

CHUNK_ROWS = 32
GROUPS = CHUNK_ROWS // LANES
NUM_CHUNKS = ROWS_PER_WORKER // CHUNK_ROWS
CHUNK_WORDS = CHUNK_ROWS * X_DIM

SUPER_ROWS = 128
CHUNKS_PER_SUPER = SUPER_ROWS // CHUNK_ROWS

_STARTS = tuple(range(0, X_DIM - LANES, LANES)) + (X_DIM - LANES,)
NUM_STRANDS = 3
_STRANDS = (_STARTS[0:21], _STARTS[21:42], _STARTS[42:63])

T_STRIDE = LANES + 1

_BIG = 2**30


def _row_scan(xbuf, rbase):
    ms = []
    idxs = []
    for strand in _STRANDS:
        m = jnp.full((LANES,), -jnp.inf, jnp.float32)
        i = jnp.zeros((LANES,), jnp.int32)
        for start in strand:
            v = xbuf[pl.ds(rbase + start, LANES)]
            cond = v > m
            m = jnp.maximum(m, v)
            i = jnp.where(cond, jnp.int32(start), i)
        ms.append(m)
        idxs.append(i)
    m, i = ms[0], idxs[0]
    for k in range(1, NUM_STRANDS):
        cond = ms[k] > m
        m = jnp.where(cond, ms[k], m)
        i = jnp.where(cond, idxs[k], i)
    return m, i


def _group_argmax(xbuf, mbuf, gbuf, lane, grp):
    def row_body(r, _):
        m, i = _row_scan(xbuf, (grp * LANES + r) * X_DIM)
        mbuf[r, pl.ds(0, LANES)] = m
        gbuf[r, pl.ds(0, LANES)] = i + lane
        return _

    lax.fori_loop(0, LANES, row_body, None)

    tms = []
    tgs = []
    for l in range(LANES):
        col = jnp.full((LANES,), l, jnp.int32)
        tms.append(plsc.load_gather(mbuf, [lane, col]))
        tgs.append(plsc.load_gather(gbuf, [lane, col]))
    m = tms[0]
    for l in range(1, LANES):
        m = jnp.maximum(m, tms[l])
    cand = jnp.full((LANES,), _BIG, jnp.int32)
    for l in range(LANES):
        cand = jnp.minimum(cand, jnp.where(tms[l] == m, tgs[l], jnp.int32(_BIG)))
    return cand


def _tec_body(x_hbm, table_hbm, out_hbm, xbuf0, xbuf1, mbuf, gbuf, idxbuf,
              rowsbuf, sem0, sem1, semg):
    wid = lax.axis_index("s") * NUM_CORES + lax.axis_index("c")
    base = wid * ROWS_PER_WORKER
    lane = lax.iota(jnp.int32, LANES)

    def x_slice(g):
        return x_hbm.at[pl.ds((base + g * CHUNK_ROWS) * X_DIM, CHUNK_WORDS)]

    def finish_chunk(g, xbuf):
        q = lax.rem(g, CHUNKS_PER_SUPER)
        for grp in range(GROUPS):
            idx16 = _group_argmax(xbuf, mbuf, gbuf, lane, grp)
            idxbuf[pl.ds(q * CHUNK_ROWS + grp * LANES, LANES)] = idx16

        @pl.when(q == CHUNKS_PER_SUPER - 1)
        def _flush():
            pltpu.async_copy(table_hbm.at[idxbuf], rowsbuf, semg).wait()
            row0 = base + (g + 1 - CHUNKS_PER_SUPER) * CHUNK_ROWS
            pltpu.sync_copy(rowsbuf, out_hbm.at[pl.ds(row0, SUPER_ROWS)])

    pltpu.async_copy(x_slice(0), xbuf0, sem0)

    def pair_body(h, _):
        g0 = 2 * h
        pltpu.async_copy(x_slice(g0 + 1), xbuf1, sem1)
        pltpu.make_async_copy(x_slice(g0), xbuf0, sem0).wait()
        finish_chunk(g0, xbuf0)

        @pl.when(h + 1 < NUM_CHUNKS // 2)
        def _prefetch():
            pltpu.async_copy(x_slice(g0 + 2), xbuf0, sem0)

        pltpu.make_async_copy(x_slice(g0 + 1), xbuf1, sem1).wait()
        finish_chunk(g0 + 1, xbuf1)
        return _

    lax.fori_loop(0, NUM_CHUNKS // 2, pair_body, None)


@jax.jit
def kernel(x, twohot_table):
    mesh = plsc.VectorSubcoreMesh(core_axis_name="c", subcore_axis_name="s")
    run = pl.kernel(
        _tec_body,
        out_type=jax.ShapeDtypeStruct((BATCH, XC_DIM), jnp.float32),
        mesh=mesh,
        scratch_types=[
            pltpu.VMEM((CHUNK_WORDS,), jnp.float32),
            pltpu.VMEM((CHUNK_WORDS,), jnp.float32),
            pltpu.VMEM((LANES, T_STRIDE), jnp.float32),
            pltpu.VMEM((LANES, T_STRIDE), jnp.int32),
            pltpu.VMEM((SUPER_ROWS,), jnp.int32),
            pltpu.VMEM((SUPER_ROWS, XC_DIM), jnp.float32),
            pltpu.SemaphoreType.DMA,
            pltpu.SemaphoreType.DMA,
            pltpu.SemaphoreType.DMA,
        ],
        compiler_params=pltpu.CompilerParams(
            use_tc_tiling_on_sc=False, needs_layout_passes=False
        ),
    )
    return run(x.reshape(-1), twohot_table)


# speedup vs baseline: 1.8941x; 1.4235x over previous
import functools

import jax
import jax.numpy as jnp
from jax import lax
from jax.experimental import pallas as pl
from jax.experimental.pallas import tpu as pltpu
from jax.experimental.pallas import tpu_sc as plsc

BATCH = 16384
X_DIM = 1000
XC_DIM = 64
LANES = 16

NUM_CORES = 2
NUM_SUBCORES = 16
NUM_WORKERS = NUM_CORES * NUM_SUBCORES
ROWS_PER_WORKER = BATCH // NUM_WORKERS

TC_BLOCK = 512

SUPER_ROWS = 128
SUPERS_PER_WORKER = ROWS_PER_WORKER // SUPER_ROWS


def _tc_argmax_body(x_ref, idx_ref):
    xb = x_ref[...]
    m = jnp.max(xb, axis=1, keepdims=True)
    io = lax.broadcasted_iota(jnp.int32, xb.shape, 1)
    cand = jnp.where(xb == m, io, jnp.int32(X_DIM))
    idx_ref[...] = jnp.min(cand, axis=1)


def _tc_argmax(x):
    return pl.pallas_call(
        _tc_argmax_body,
        grid=(BATCH // TC_BLOCK,),
        in_specs=[pl.BlockSpec((TC_BLOCK, X_DIM), lambda i: (i, 0))],
        out_specs=pl.BlockSpec((TC_BLOCK,), lambda i: (i,)),
        out_shape=jax.ShapeDtypeStruct((BATCH,), jnp.int32),
    )(x)


def _sc_gather_body(table_hbm, idx_hbm, out_hbm, idxbuf, rowsbuf, semg):
    wid = lax.axis_index("s") * NUM_CORES + lax.axis_index("c")
    base = wid * ROWS_PER_WORKER

    for s in range(SUPERS_PER_WORKER):
        row0 = base + s * SUPER_ROWS
        pltpu.sync_copy(idx_hbm.at[pl.ds(row0, SUPER_ROWS)], idxbuf)
        pltpu.async_copy(table_hbm.at[idxbuf], rowsbuf, semg).wait()
        pltpu.sync_copy(rowsbuf, out_hbm.at[pl.ds(row0, SUPER_ROWS)])


def _sc_gather(table, idx):
    mesh = plsc.VectorSubcoreMesh(core_axis_name="c", subcore_axis_name="s")
    run = pl.kernel(
        _sc_gather_body,
        out_type=jax.ShapeDtypeStruct((BATCH, XC_DIM), jnp.float32),
        mesh=mesh,
        scratch_types=[
            pltpu.VMEM((SUPER_ROWS,), jnp.int32),
            pltpu.VMEM((SUPER_ROWS, XC_DIM), jnp.float32),
            pltpu.SemaphoreType.DMA,
        ],
        compiler_params=pltpu.CompilerParams(
            use_tc_tiling_on_sc=False, needs_layout_passes=False
        ),
    )
    return run(table, idx)


@jax.jit
def kernel(x, twohot_table):
    idx = _tc_argmax(x)
    return _sc_gather(twohot_table, idx)
